# baseline (speedup 1.0000x reference)
import functools

import jax
import jax.numpy as jnp
from jax import lax
from jax.experimental import pallas as pl
from jax.experimental.pallas import tpu as pltpu
from jax.experimental.pallas import tpu_sc as plsc

N = 10000
NPAD = 10240
E = 320000
NC, NS, LANES = 2, 16, 16
NT = NC * NS
B = 128
K = 80
EPT = K * B
EPAD = NT * EPT
RPT = NPAD // NS

_mesh = plsc.VectorSubcoreMesh(
    core_axis_name="c", subcore_axis_name="s", num_cores=NC, num_subcores=NS)


NB = 8
GL = 6


def _edge_loop(hs_hbm, srcv, dstv, bufs, acc, gsem, ssem):
    for p in range(GL):
        pltpu.async_copy(hs_hbm.at[srcv.at[p]], bufs[p], gsem[p])

    def roundn(jo, carry):
        for r in range(NB):
            idx = NB * jo + r
            rn = (r + GL) % NB
            pltpu.make_async_copy(
                hs_hbm.at[srcv.at[idx]], bufs[r], gsem[r]).wait()
            pltpu.async_copy(
                bufs[r], acc.at[dstv.at[idx]], ssem[r], add=True)

            @pl.when(idx >= NB - GL)
            def _():
                pltpu.make_async_copy(
                    bufs[rn], acc.at[dstv.at[idx - (NB - GL)]],
                    ssem[rn]).wait()

            @pl.when(idx + GL < K)
            def _():
                pltpu.async_copy(
                    hs_hbm.at[srcv.at[idx + GL]], bufs[rn], gsem[rn])
        return carry

    lax.fori_loop(0, K // NB, roundn, 0)
    for t in range(K - (NB - GL), K):
        pltpu.make_async_copy(bufs[t % NB],
                              acc.at[dstv.at[t]], ssem[t % NB]).wait()


@functools.partial(
    pl.kernel,
    out_type=[jax.ShapeDtypeStruct((NC, NS, RPT, 32), jnp.float32),
              jax.ShapeDtypeStruct((NPAD, 32), jnp.float32),
              jax.ShapeDtypeStruct((NPAD,), jnp.float32)],
    mesh=_mesh,
    scratch_types=[
        pltpu.VMEM((K, B), jnp.int32),
        pltpu.VMEM((K, B), jnp.int32),
        pltpu.VMEM((2 * K, B), jnp.int32),
        [pltpu.VMEM((B, 32), jnp.float32)] * NB,
        pltpu.VMEM((RPT, 32), jnp.float32),
        pltpu.VMEM((RPT,), jnp.float32),
        pltpu.VMEM((RPT,), jnp.float32),
        pltpu.VMEM((B,), jnp.float32),
        pltpu.VMEM_SHARED((NPAD,), jnp.float32),
        pltpu.VMEM_SHARED((NPAD, 32), jnp.float32),
        [pltpu.SemaphoreType.DMA] * NB,
        [pltpu.SemaphoreType.DMA] * NB,
        pltpu.SemaphoreType.DMA,
    ],
    compiler_params=pltpu.CompilerParams(use_tc_tiling_on_sc=False,
                                        needs_layout_passes=False),
)
def _layer1_kernel(h1_hbm, srcT_hbm, dstT_hbm, zer_hbm,
                   out_hbm, hs1_hbm, dinv_hbm,
                   srcv, dstv, dsth, bufs, hv, degv, dinvv, onesv,
                   degacc, acc, gsem, ssem, hsem):
    c = lax.axis_index("c")
    s = lax.axis_index("s")
    wid = s * NC + c
    r0 = s * RPT
    one = jnp.ones((LANES,), jnp.float32)
    zero = jnp.zeros((LANES,), jnp.float32)
    for i in range(0, B, LANES):
        onesv[pl.ds(i, LANES)] = one
    for i in range(0, RPT, LANES):
        degv[pl.ds(i, LANES)] = zero
    pltpu.sync_copy(degv, degacc.at[pl.ds(r0, RPT)])
    pltpu.sync_copy(zer_hbm.at[pl.ds(r0, RPT)], acc.at[pl.ds(r0, RPT)])
    pltpu.sync_copy(dstT_hbm.at[2 * s], dsth.at[pl.ds(0, K)])
    pltpu.sync_copy(dstT_hbm.at[2 * s + 1], dsth.at[pl.ds(K, K)])
    pltpu.sync_copy(srcT_hbm.at[wid], srcv)
    pltpu.sync_copy(dstT_hbm.at[wid], dstv)
    plsc.subcore_barrier()

    def fire(j, carry):
        pltpu.async_copy(onesv, degacc.at[dsth.at[j]], hsem, add=True)
        return carry

    lax.fori_loop(0, 2 * K, fire, 0)

    def drain(j, carry):
        pltpu.make_async_copy(onesv, degacc.at[dsth.at[j]], hsem).wait()
        return carry

    lax.fori_loop(0, 2 * K, drain, 0)
    plsc.subcore_barrier()

    pltpu.sync_copy(degacc.at[pl.ds(r0, RPT)], degv)
    magic = jnp.full((LANES,), 0x5F3759DF, jnp.int32)
    for i in range(0, RPT, LANES):
        d = degv[pl.ds(i, LANES)] + 1.0
        y = plsc.bitcast(
            magic - lax.shift_right_logical(plsc.bitcast(d, jnp.int32), 1),
            jnp.float32)
        for _ in range(3):
            y = y * (1.5 - 0.5 * d * y * y)
        dinvv[pl.ds(i, LANES)] = y
    pltpu.sync_copy(dinvv, dinv_hbm.at[pl.ds(r0, RPT)])

    pltpu.sync_copy(h1_hbm.at[pl.ds(r0, RPT)], hv)

    def scale_group(g, carry):
        dv = dinvv[pl.ds(g * LANES, LANES)]
        for j in range(LANES):
            r = g * LANES + j
            sc = dv[j]
            hv[r, pl.ds(0, LANES)] = hv[r, pl.ds(0, LANES)] * sc
            hv[r, pl.ds(LANES, LANES)] = hv[r, pl.ds(LANES, LANES)] * sc
        return carry

    lax.fori_loop(0, RPT // LANES, scale_group, 0)
    pltpu.sync_copy(hv, hs1_hbm.at[pl.ds(r0, RPT)])
    plsc.subcore_barrier()

    _edge_loop(hs1_hbm, srcv, dstv, bufs, acc, gsem, ssem)
    plsc.subcore_barrier()
    pltpu.sync_copy(acc.at[pl.ds(r0, RPT)], out_hbm.at[c, s])


def _make_agg(D):
    @functools.partial(
        pl.kernel,
        out_type=jax.ShapeDtypeStruct((NC, NS, RPT, D), jnp.float32),
        mesh=_mesh,
        scratch_types=[
            pltpu.VMEM((K, B), jnp.int32),
            pltpu.VMEM((K, B), jnp.int32),
            [pltpu.VMEM((B, D), jnp.float32)] * 8,
            pltpu.VMEM_SHARED((NPAD, D), jnp.float32),
            [pltpu.SemaphoreType.DMA] * 8,
            [pltpu.SemaphoreType.DMA] * 8,
        ],
        compiler_params=pltpu.CompilerParams(use_tc_tiling_on_sc=False),
    )
    def agg(hs_hbm, srcT_hbm, dstT_hbm, zer_hbm, out_hbm,
            srcv, dstv, bufs, acc, gsem, ssem):
        c = lax.axis_index("c")
        s = lax.axis_index("s")
        wid = s * NC + c
        pltpu.sync_copy(zer_hbm.at[pl.ds(s * RPT, RPT)],
                        acc.at[pl.ds(s * RPT, RPT)])
        pltpu.sync_copy(srcT_hbm.at[wid], srcv)
        pltpu.sync_copy(dstT_hbm.at[wid], dstv)
        plsc.subcore_barrier()
        _edge_loop(hs_hbm, srcv, dstv, bufs, acc, gsem, ssem)
        plsc.subcore_barrier()
        pltpu.sync_copy(acc.at[pl.ds(s * RPT, RPT)], out_hbm.at[c, s])

    return agg


_agg8 = _make_agg(8)


def _k2a_body(x_ref, w1_ref, h1_ref):
    h1_ref[...] = jnp.dot(x_ref[...], w1_ref[...],
                          preferred_element_type=jnp.float32)


def _k4_body(pa_ref, pb_ref, hs1_ref, dinv_ref, b1_ref, w2_ref, hs2_ref):
    dinv = dinv_ref[...]
    full = pa_ref[...] + pb_ref[...] + hs1_ref[...]
    h = jnp.maximum(full * dinv + b1_ref[...], 0.0)
    h2 = jnp.dot(h, w2_ref[...], preferred_element_type=jnp.float32)
    hs2_ref[...] = h2 * dinv


def _k6_body(pa_ref, pb_ref, hs2_ref, dinv_ref, b2_ref, out_ref):
    logits = ((pa_ref[...] + pb_ref[...] + hs2_ref[...]) * dinv_ref[...]
              + b2_ref[...])
    m = jnp.max(logits, axis=1, keepdims=True)
    e = jnp.exp(logits - m)
    out_ref[...] = (e / jnp.sum(e, axis=1, keepdims=True))[:, :7]


@jax.jit
def kernel(x, edge_index, W1, b1, W2, b2):
    f32 = jnp.float32
    src = edge_index[0].astype(jnp.int32)
    dst = edge_index[1].astype(jnp.int32)
    npad_e = EPAD - E
    pad_i = jnp.arange(npad_e, dtype=jnp.int32)
    srcT = jnp.concatenate([src, pad_i % N]).reshape(NT, K, B)
    dstT = jnp.concatenate([dst, N + pad_i % (NPAD - N)]).reshape(NT, K, B)

    xp = jnp.concatenate([x, jnp.zeros((NPAD - N, x.shape[1]), f32)])
    w2p = jnp.concatenate([W2, jnp.zeros((W2.shape[0], 1), f32)], axis=1)
    b1r = b1.reshape(1, -1)
    b2r = jnp.concatenate([b2, jnp.full((1,), -1e30, f32)]).reshape(1, 8)
    zer32 = jnp.zeros((NPAD, 32), f32)
    zer8 = jnp.zeros((NPAD, 8), f32)

    h1 = pl.pallas_call(
        _k2a_body,
        out_shape=jax.ShapeDtypeStruct((NPAD, 32), f32),
    )(xp, W1)
    p1, hs1, dinv1 = _layer1_kernel(h1, srcT, dstT, zer32)
    dinv = dinv1.reshape(NPAD, 1)

    hs2 = pl.pallas_call(
        _k4_body,
        out_shape=jax.ShapeDtypeStruct((NPAD, 8), f32),
    )(p1[0].reshape(NPAD, 32), p1[1].reshape(NPAD, 32), hs1, dinv, b1r, w2p)

    p2 = _agg8(hs2, srcT, dstT, zer8)
    out = pl.pallas_call(
        _k6_body,
        out_shape=jax.ShapeDtypeStruct((NPAD, 7), f32),
    )(p2[0].reshape(NPAD, 8), p2[1].reshape(NPAD, 8), hs2, dinv, b2r)
    return out[:N]


# speedup vs baseline: 1.1033x; 1.1033x over previous
import functools

import jax
import jax.numpy as jnp
from jax import lax
from jax.experimental import pallas as pl
from jax.experimental.pallas import tpu as pltpu
from jax.experimental.pallas import tpu_sc as plsc

N = 10000
NPAD = 10240
E = 320000
NC, NS, LANES = 2, 16, 16
NT = NC * NS
B = 128
K = 80
EPT = K * B
EPAD = NT * EPT
RPT = NPAD // NS

_mesh = plsc.VectorSubcoreMesh(
    core_axis_name="c", subcore_axis_name="s", num_cores=NC, num_subcores=NS)


NB = 8
GL = 6


def _edge_loop(hs_hbm, srcv, dstv, bufs, acc, gsem, ssem):
    for p in range(GL):
        pltpu.async_copy(hs_hbm.at[srcv.at[p]], bufs[p], gsem[p])

    def roundn(jo, carry):
        for r in range(NB):
            idx = NB * jo + r
            rn = (r + GL) % NB
            pltpu.make_async_copy(
                hs_hbm.at[srcv.at[idx]], bufs[r], gsem[r]).wait()
            pltpu.async_copy(
                bufs[r], acc.at[dstv.at[idx]], ssem[r], add=True)

            @pl.when(idx >= NB - GL)
            def _():
                pltpu.make_async_copy(
                    bufs[rn], acc.at[dstv.at[idx - (NB - GL)]],
                    ssem[rn]).wait()

            @pl.when(idx + GL < K)
            def _():
                pltpu.async_copy(
                    hs_hbm.at[srcv.at[idx + GL]], bufs[rn], gsem[rn])
        return carry

    lax.fori_loop(0, K // NB, roundn, 0)
    for t in range(K - (NB - GL), K):
        pltpu.make_async_copy(bufs[t % NB],
                              acc.at[dstv.at[t]], ssem[t % NB]).wait()


@functools.partial(
    pl.kernel,
    out_type=[jax.ShapeDtypeStruct((NC, NPAD, 32), jnp.float32),
              jax.ShapeDtypeStruct((NPAD, 32), jnp.float32),
              jax.ShapeDtypeStruct((NPAD, LANES), jnp.float32)],
    mesh=_mesh,
    scratch_types=[
        pltpu.VMEM((K, B), jnp.int32),
        pltpu.VMEM((K, B), jnp.int32),
        pltpu.VMEM((2 * K, B), jnp.int32),
        [pltpu.VMEM((B, 32), jnp.float32)] * NB,
        pltpu.VMEM((RPT, 32), jnp.float32),
        pltpu.VMEM((RPT,), jnp.float32),
        pltpu.VMEM((RPT,), jnp.float32),
        pltpu.VMEM((RPT, LANES), jnp.float32),
        pltpu.VMEM((B,), jnp.float32),
        pltpu.VMEM_SHARED((NPAD,), jnp.float32),
        pltpu.VMEM_SHARED((NPAD, 32), jnp.float32),
        [pltpu.SemaphoreType.DMA] * NB,
        [pltpu.SemaphoreType.DMA] * NB,
        pltpu.SemaphoreType.DMA,
    ],
    compiler_params=pltpu.CompilerParams(use_tc_tiling_on_sc=False,
                                        needs_layout_passes=False),
)
def _layer1_kernel(h1_hbm, ei_hbm, zer_hbm,
                   out_hbm, hs1_hbm, dinv_hbm,
                   srcv, dstv, dsth, bufs, hv, degv, dinvv, dinvb, onesv,
                   degacc, acc, gsem, ssem, hsem):
    c = lax.axis_index("c")
    s = lax.axis_index("s")
    wid = s * NC + c
    r0 = s * RPT
    one = jnp.ones((LANES,), jnp.float32)
    zero = jnp.zeros((LANES,), jnp.float32)
    for i in range(0, B, LANES):
        onesv[pl.ds(i, LANES)] = one
    for i in range(0, RPT, LANES):
        degv[pl.ds(i, LANES)] = zero
    pltpu.sync_copy(degv, degacc.at[pl.ds(r0, RPT)])
    pltpu.sync_copy(zer_hbm.at[pl.ds(r0, RPT)], acc.at[pl.ds(r0, RPT)])
    pltpu.sync_copy(ei_hbm.at[1, 2 * s], dsth.at[pl.ds(0, K)])
    pltpu.sync_copy(ei_hbm.at[1, 2 * s + 1], dsth.at[pl.ds(K, K)])
    pltpu.sync_copy(ei_hbm.at[0, wid], srcv)
    pltpu.sync_copy(ei_hbm.at[1, wid], dstv)
    plsc.subcore_barrier()

    def fire(j, carry):
        pltpu.async_copy(onesv, degacc.at[dsth.at[j]], hsem, add=True)
        return carry

    lax.fori_loop(0, 2 * K, fire, 0)

    def drain(j, carry):
        pltpu.make_async_copy(onesv, degacc.at[dsth.at[j]], hsem).wait()
        return carry

    lax.fori_loop(0, 2 * K, drain, 0)
    plsc.subcore_barrier()

    pltpu.sync_copy(degacc.at[pl.ds(r0, RPT)], degv)
    magic = jnp.full((LANES,), 0x5F3759DF, jnp.int32)
    for i in range(0, RPT, LANES):
        d = degv[pl.ds(i, LANES)] + 1.0
        y = plsc.bitcast(
            magic - lax.shift_right_logical(plsc.bitcast(d, jnp.int32), 1),
            jnp.float32)
        for _ in range(3):
            y = y * (1.5 - 0.5 * d * y * y)
        dinvv[pl.ds(i, LANES)] = y

    pltpu.sync_copy(h1_hbm.at[pl.ds(r0, RPT)], hv)

    def scale_group(g, carry):
        dv = dinvv[pl.ds(g * LANES, LANES)]
        for j in range(LANES):
            r = g * LANES + j
            sc = dv[j]
            hv[r, pl.ds(0, LANES)] = hv[r, pl.ds(0, LANES)] * sc
            hv[r, pl.ds(LANES, LANES)] = hv[r, pl.ds(LANES, LANES)] * sc
            dinvb[r, pl.ds(0, LANES)] = jnp.full((LANES,), sc, jnp.float32)
        return carry

    lax.fori_loop(0, RPT // LANES, scale_group, 0)
    pltpu.sync_copy(dinvb, dinv_hbm.at[pl.ds(r0, RPT)])
    pltpu.sync_copy(hv, hs1_hbm.at[pl.ds(r0, RPT)])
    plsc.subcore_barrier()

    _edge_loop(hs1_hbm, srcv, dstv, bufs, acc, gsem, ssem)
    plsc.subcore_barrier()
    pltpu.sync_copy(acc.at[pl.ds(r0, RPT)], out_hbm.at[c, pl.ds(r0, RPT)])


@functools.partial(
    pl.kernel,
    out_type=jax.ShapeDtypeStruct((NC, NPAD, 8), jnp.float32),
    mesh=_mesh,
    scratch_types=[
        pltpu.VMEM((K, B), jnp.int32),
        pltpu.VMEM((K, B), jnp.int32),
        [pltpu.VMEM((B, 8), jnp.float32)] * NB,
        pltpu.VMEM_SHARED((NPAD, 8), jnp.float32),
        [pltpu.SemaphoreType.DMA] * NB,
        [pltpu.SemaphoreType.DMA] * NB,
    ],
    compiler_params=pltpu.CompilerParams(use_tc_tiling_on_sc=False),
)
def _agg8(hs_hbm, ei_hbm, zer_hbm, out_hbm,
          srcv, dstv, bufs, acc, gsem, ssem):
    c = lax.axis_index("c")
    s = lax.axis_index("s")
    wid = s * NC + c
    r0 = s * RPT
    pltpu.sync_copy(zer_hbm.at[pl.ds(r0, RPT)], acc.at[pl.ds(r0, RPT)])
    pltpu.sync_copy(ei_hbm.at[0, wid], srcv)
    pltpu.sync_copy(ei_hbm.at[1, wid], dstv)
    plsc.subcore_barrier()
    _edge_loop(hs_hbm, srcv, dstv, bufs, acc, gsem, ssem)
    plsc.subcore_barrier()
    pltpu.sync_copy(acc.at[pl.ds(r0, RPT)], out_hbm.at[c, pl.ds(r0, RPT)])


_RB = 1024
_GRID = NPAD // _RB


def _k2a_body(x_ref, w1_ref, h1_ref):
    h1_ref[...] = jnp.dot(x_ref[...], w1_ref[...],
                          preferred_element_type=jnp.float32)


def _k4_body(p_ref, hs1_ref, dinv_ref, b1_ref, w2_ref, hs2_ref):
    dinv = dinv_ref[...][:, 0:1]
    full = p_ref[0] + p_ref[1] + hs1_ref[...]
    h = jnp.maximum(full * dinv + b1_ref[...], 0.0)
    h2 = jnp.dot(h, w2_ref[...], preferred_element_type=jnp.float32)
    hs2_ref[...] = h2 * dinv


def _k6_body(p_ref, hs2_ref, dinv_ref, b2_ref, out_ref):
    logits = ((p_ref[0] + p_ref[1] + hs2_ref[...]) * dinv_ref[...][:, 0:1]
              + b2_ref[...])
    m = jnp.max(logits, axis=1, keepdims=True)
    e = jnp.exp(logits - m)
    out_ref[...] = (e / jnp.sum(e, axis=1, keepdims=True))[:, :7]


@jax.jit
def kernel(x, edge_index, W1, b1, W2, b2):
    f32 = jnp.float32
    ei = edge_index.astype(jnp.int32)
    npad_e = EPAD - E
    pad_i = jnp.arange(npad_e, dtype=jnp.int32)
    pads = jnp.stack([pad_i % N, N + pad_i % (NPAD - N)])
    eiT = jnp.concatenate([ei, pads], axis=1).reshape(2, NT, K, B)

    xp = jnp.concatenate([x, jnp.zeros((NPAD - N, x.shape[1]), f32)])
    w2p = jnp.concatenate([W2, jnp.zeros((W2.shape[0], 1), f32)], axis=1)
    b1r = b1.reshape(1, -1)
    b2r = jnp.concatenate([b2, jnp.full((1,), -1e30, f32)]).reshape(1, 8)
    zer32 = jnp.zeros((NPAD, 32), f32)
    zer8 = jnp.zeros((NPAD, 8), f32)

    h1 = pl.pallas_call(
        _k2a_body,
        grid=(_GRID,),
        in_specs=[pl.BlockSpec((_RB, 128), lambda i: (i, 0)),
                  pl.BlockSpec((128, 32), lambda i: (0, 0))],
        out_specs=pl.BlockSpec((_RB, 32), lambda i: (i, 0)),
        out_shape=jax.ShapeDtypeStruct((NPAD, 32), f32),
    )(xp, W1)
    p1, hs1, dinv16 = _layer1_kernel(h1, eiT, zer32)

    hs2 = pl.pallas_call(
        _k4_body,
        grid=(_GRID,),
        in_specs=[pl.BlockSpec((NC, _RB, 32), lambda i: (0, i, 0)),
                  pl.BlockSpec((_RB, 32), lambda i: (i, 0)),
                  pl.BlockSpec((_RB, LANES), lambda i: (i, 0)),
                  pl.BlockSpec((1, 32), lambda i: (0, 0)),
                  pl.BlockSpec((32, 8), lambda i: (0, 0))],
        out_specs=pl.BlockSpec((_RB, 8), lambda i: (i, 0)),
        out_shape=jax.ShapeDtypeStruct((NPAD, 8), f32),
    )(p1, hs1, dinv16, b1r, w2p)

    p2 = _agg8(hs2, eiT, zer8)
    out = pl.pallas_call(
        _k6_body,
        grid=(_GRID,),
        in_specs=[pl.BlockSpec((NC, _RB, 8), lambda i: (0, i, 0)),
                  pl.BlockSpec((_RB, 8), lambda i: (i, 0)),
                  pl.BlockSpec((_RB, LANES), lambda i: (i, 0)),
                  pl.BlockSpec((1, 8), lambda i: (0, 0))],
        out_specs=pl.BlockSpec((_RB, 7), lambda i: (i, 0)),
        out_shape=jax.ShapeDtypeStruct((NPAD, 7), f32),
    )(p2, hs2, dinv16, b2r)
    return out[:N]


# speedup vs baseline: 1.1572x; 1.0488x over previous
import functools

import jax
import jax.numpy as jnp
from jax import lax
from jax.experimental import pallas as pl
from jax.experimental.pallas import tpu as pltpu
from jax.experimental.pallas import tpu_sc as plsc

N = 10000
NPAD = 10240
E = 320000
NC, NS, LANES = 2, 16, 16
NT = NC * NS
B = 128
K = 80
EPT = K * B
EPAD = NT * EPT
RPT = NPAD // NS

_mesh = plsc.VectorSubcoreMesh(
    core_axis_name="c", subcore_axis_name="s", num_cores=NC, num_subcores=NS)


@functools.partial(
    pl.kernel,
    out_type=jax.ShapeDtypeStruct((NC, NPAD), jnp.float32),
    mesh=_mesh,
    scratch_types=[
        pltpu.VMEM((K, B), jnp.int32),
        pltpu.VMEM((B,), jnp.float32),
        pltpu.VMEM((RPT,), jnp.float32),
        pltpu.VMEM_SHARED((NPAD,), jnp.float32),
        pltpu.SemaphoreType.DMA,
    ],
    compiler_params=pltpu.CompilerParams(use_tc_tiling_on_sc=False),
)
def _deg_kernel(ei_hbm, out_hbm, dstv, onesv, zbuf, acc, sem):
    c = lax.axis_index("c")
    s = lax.axis_index("s")
    wid = s * NC + c
    one = jnp.ones((LANES,), jnp.float32)
    zero = jnp.zeros((LANES,), jnp.float32)
    for i in range(0, B, LANES):
        onesv[pl.ds(i, LANES)] = one
    for i in range(0, RPT, LANES):
        zbuf[pl.ds(i, LANES)] = zero
    pltpu.sync_copy(zbuf, acc.at[pl.ds(s * RPT, RPT)])
    pltpu.sync_copy(ei_hbm.at[1, wid], dstv)
    plsc.subcore_barrier()

    def fire(j, carry):
        pltpu.async_copy(onesv, acc.at[dstv.at[j]], sem, add=True)
        return carry

    lax.fori_loop(0, K, fire, 0)

    def drain(j, carry):
        pltpu.make_async_copy(onesv, acc.at[dstv.at[j]], sem).wait()
        return carry

    lax.fori_loop(0, K, drain, 0)
    plsc.subcore_barrier()
    pltpu.sync_copy(acc.at[pl.ds(s * RPT, RPT)],
                    out_hbm.at[c, pl.ds(s * RPT, RPT)])


NB = 8
GL = 6


def _edge_loop(hs_hbm, srcv, dstv, bufs, acc, gsem, ssem):
    for p in range(GL):
        pltpu.async_copy(hs_hbm.at[srcv.at[p]], bufs[p], gsem[p])

    def roundn(jo, carry):
        for r in range(NB):
            idx = NB * jo + r
            rn = (r + GL) % NB
            pltpu.make_async_copy(
                hs_hbm.at[srcv.at[idx]], bufs[r], gsem[r]).wait()
            pltpu.async_copy(
                bufs[r], acc.at[dstv.at[idx]], ssem[r], add=True)

            @pl.when(idx >= NB - GL)
            def _():
                pltpu.make_async_copy(
                    bufs[rn], acc.at[dstv.at[idx - (NB - GL)]],
                    ssem[rn]).wait()

            @pl.when(idx + GL < K)
            def _():
                pltpu.async_copy(
                    hs_hbm.at[srcv.at[idx + GL]], bufs[rn], gsem[rn])
        return carry

    lax.fori_loop(0, K // NB, roundn, 0)
    for t in range(K - (NB - GL), K):
        pltpu.make_async_copy(bufs[t % NB],
                              acc.at[dstv.at[t]], ssem[t % NB]).wait()


@functools.partial(
    pl.kernel,
    out_type=[jax.ShapeDtypeStruct((NC, NPAD, 32), jnp.float32),
              jax.ShapeDtypeStruct((NPAD, 32), jnp.float32),
              jax.ShapeDtypeStruct((NPAD, LANES), jnp.float32)],
    mesh=_mesh,
    scratch_types=[
        pltpu.VMEM((K, B), jnp.int32),
        pltpu.VMEM((K, B), jnp.int32),
        [pltpu.VMEM((B, 32), jnp.float32)] * NB,
        pltpu.VMEM((RPT, 32), jnp.float32),
        pltpu.VMEM((RPT,), jnp.float32),
        pltpu.VMEM((RPT,), jnp.float32),
        pltpu.VMEM((RPT,), jnp.float32),
        pltpu.VMEM((RPT, LANES), jnp.float32),
        pltpu.VMEM_SHARED((NPAD, 32), jnp.float32),
        [pltpu.SemaphoreType.DMA] * NB,
        [pltpu.SemaphoreType.DMA] * NB,
    ],
    compiler_params=pltpu.CompilerParams(use_tc_tiling_on_sc=False,
                                        needs_layout_passes=False),
)
def _layer1_kernel(h1_hbm, ei_hbm, degp_hbm,
                   out_hbm, hs1_hbm, dinv_hbm,
                   srcv, dstv, bufs, hv, degv, degv2, dinvv, dinvb,
                   acc, gsem, ssem):
    c = lax.axis_index("c")
    s = lax.axis_index("s")
    wid = s * NC + c
    r0 = s * RPT
    zero = jnp.zeros((LANES,), jnp.float32)
    def zrow(r, carry):
        hv[r, pl.ds(0, LANES)] = zero
        hv[r, pl.ds(LANES, LANES)] = zero
        return carry

    lax.fori_loop(0, RPT, zrow, 0)
    pltpu.sync_copy(hv, acc.at[pl.ds(r0, RPT)])
    pltpu.sync_copy(ei_hbm.at[0, wid], srcv)
    pltpu.sync_copy(ei_hbm.at[1, wid], dstv)

    pltpu.sync_copy(degp_hbm.at[0, pl.ds(r0, RPT)], degv)
    pltpu.sync_copy(degp_hbm.at[1, pl.ds(r0, RPT)], degv2)
    magic = jnp.full((LANES,), 0x5F3759DF, jnp.int32)
    for i in range(0, RPT, LANES):
        d = degv[pl.ds(i, LANES)] + degv2[pl.ds(i, LANES)] + 1.0
        y = plsc.bitcast(
            magic - lax.shift_right_logical(plsc.bitcast(d, jnp.int32), 1),
            jnp.float32)
        for _ in range(3):
            y = y * (1.5 - 0.5 * d * y * y)
        dinvv[pl.ds(i, LANES)] = y

    pltpu.sync_copy(h1_hbm.at[pl.ds(r0, RPT)], hv)

    def scale_group(g, carry):
        dv = dinvv[pl.ds(g * LANES, LANES)]
        for j in range(LANES):
            r = g * LANES + j
            sc = dv[j]
            hv[r, pl.ds(0, LANES)] = hv[r, pl.ds(0, LANES)] * sc
            hv[r, pl.ds(LANES, LANES)] = hv[r, pl.ds(LANES, LANES)] * sc
            dinvb[r, pl.ds(0, LANES)] = jnp.full((LANES,), sc, jnp.float32)
        return carry

    lax.fori_loop(0, RPT // LANES, scale_group, 0)
    pltpu.sync_copy(dinvb, dinv_hbm.at[pl.ds(r0, RPT)])
    pltpu.sync_copy(hv, hs1_hbm.at[pl.ds(r0, RPT)])
    plsc.subcore_barrier()

    _edge_loop(hs1_hbm, srcv, dstv, bufs, acc, gsem, ssem)
    plsc.subcore_barrier()
    pltpu.sync_copy(acc.at[pl.ds(r0, RPT)], out_hbm.at[c, pl.ds(r0, RPT)])


@functools.partial(
    pl.kernel,
    out_type=jax.ShapeDtypeStruct((NC, NPAD, 8), jnp.float32),
    mesh=_mesh,
    scratch_types=[
        pltpu.VMEM((K, B), jnp.int32),
        pltpu.VMEM((K, B), jnp.int32),
        [pltpu.VMEM((B, 8), jnp.float32)] * NB,
        pltpu.VMEM_SHARED((NPAD, 8), jnp.float32),
        [pltpu.SemaphoreType.DMA] * NB,
        [pltpu.SemaphoreType.DMA] * NB,
    ],
    compiler_params=pltpu.CompilerParams(use_tc_tiling_on_sc=False),
)
def _agg8(hs_hbm, ei_hbm, zer_hbm, out_hbm,
          srcv, dstv, bufs, acc, gsem, ssem):
    c = lax.axis_index("c")
    s = lax.axis_index("s")
    wid = s * NC + c
    r0 = s * RPT
    pltpu.sync_copy(zer_hbm.at[pl.ds(r0, RPT)], acc.at[pl.ds(r0, RPT)])
    pltpu.sync_copy(ei_hbm.at[0, wid], srcv)
    pltpu.sync_copy(ei_hbm.at[1, wid], dstv)
    plsc.subcore_barrier()
    _edge_loop(hs_hbm, srcv, dstv, bufs, acc, gsem, ssem)
    plsc.subcore_barrier()
    pltpu.sync_copy(acc.at[pl.ds(r0, RPT)], out_hbm.at[c, pl.ds(r0, RPT)])


_RB = 1024
_GRID = NPAD // _RB


def _k2a_body(x_ref, w1_ref, h1_ref):
    h1_ref[...] = jnp.dot(x_ref[...], w1_ref[...],
                          preferred_element_type=jnp.float32)


def _k4_body(p_ref, hs1_ref, dinv_ref, b1_ref, w2_ref, hs2_ref):
    dinv = dinv_ref[...][:, 0:1]
    full = p_ref[0] + p_ref[1] + hs1_ref[...]
    h = jnp.maximum(full * dinv + b1_ref[...], 0.0)
    h2 = jnp.dot(h, w2_ref[...], preferred_element_type=jnp.float32)
    hs2_ref[...] = h2 * dinv


def _k6_body(p_ref, hs2_ref, dinv_ref, b2_ref, out_ref):
    logits = ((p_ref[0] + p_ref[1] + hs2_ref[...]) * dinv_ref[...][:, 0:1]
              + b2_ref[...])
    m = jnp.max(logits, axis=1, keepdims=True)
    e = jnp.exp(logits - m)
    out_ref[...] = (e / jnp.sum(e, axis=1, keepdims=True))[:, :7]


@jax.jit
def kernel(x, edge_index, W1, b1, W2, b2):
    f32 = jnp.float32
    ei = edge_index.astype(jnp.int32)
    npad_e = EPAD - E
    pad_i = jnp.arange(npad_e, dtype=jnp.int32)
    pads = jnp.stack([pad_i % N, N + pad_i % (NPAD - N)])
    eiT = jnp.concatenate([ei, pads], axis=1).reshape(2, NT, K, B)

    xp = jnp.concatenate([x, jnp.zeros((NPAD - N, x.shape[1]), f32)])
    w2p = jnp.concatenate([W2, jnp.zeros((W2.shape[0], 1), f32)], axis=1)
    b1r = b1.reshape(1, -1)
    b2r = jnp.concatenate([b2, jnp.full((1,), -1e30, f32)]).reshape(1, 8)
    zer8 = jnp.zeros((NPAD, 8), f32)

    degp = _deg_kernel(eiT)
    h1 = pl.pallas_call(
        _k2a_body,
        grid=(_GRID,),
        in_specs=[pl.BlockSpec((_RB, 128), lambda i: (i, 0)),
                  pl.BlockSpec((128, 32), lambda i: (0, 0))],
        out_specs=pl.BlockSpec((_RB, 32), lambda i: (i, 0)),
        out_shape=jax.ShapeDtypeStruct((NPAD, 32), f32),
    )(xp, W1)
    p1, hs1, dinv16 = _layer1_kernel(h1, eiT, degp)

    hs2 = pl.pallas_call(
        _k4_body,
        grid=(_GRID,),
        in_specs=[pl.BlockSpec((NC, _RB, 32), lambda i: (0, i, 0)),
                  pl.BlockSpec((_RB, 32), lambda i: (i, 0)),
                  pl.BlockSpec((_RB, LANES), lambda i: (i, 0)),
                  pl.BlockSpec((1, 32), lambda i: (0, 0)),
                  pl.BlockSpec((32, 8), lambda i: (0, 0))],
        out_specs=pl.BlockSpec((_RB, 8), lambda i: (i, 0)),
        out_shape=jax.ShapeDtypeStruct((NPAD, 8), f32),
    )(p1, hs1, dinv16, b1r, w2p)

    p2 = _agg8(hs2, eiT, zer8)
    out = pl.pallas_call(
        _k6_body,
        grid=(_GRID,),
        in_specs=[pl.BlockSpec((NC, _RB, 8), lambda i: (0, i, 0)),
                  pl.BlockSpec((_RB, 8), lambda i: (i, 0)),
                  pl.BlockSpec((_RB, LANES), lambda i: (i, 0)),
                  pl.BlockSpec((1, 8), lambda i: (0, 0))],
        out_specs=pl.BlockSpec((_RB, 7), lambda i: (i, 0)),
        out_shape=jax.ShapeDtypeStruct((NPAD, 7), f32),
    )(p2, hs2, dinv16, b2r)
    return out[:N]


# speedup vs baseline: 1.2999x; 1.1233x over previous
import functools

import jax
import jax.numpy as jnp
from jax import lax
from jax.experimental import pallas as pl
from jax.experimental.pallas import tpu as pltpu
from jax.experimental.pallas import tpu_sc as plsc

N = 10000
NPAD = 10240
E = 320000
NC, NS, LANES = 2, 16, 16
NT = NC * NS
B = 128
K = 80
EPT = K * B
EPAD = NT * EPT
RPT = NPAD // NS

_mesh = plsc.VectorSubcoreMesh(
    core_axis_name="c", subcore_axis_name="s", num_cores=NC, num_subcores=NS)


@functools.partial(
    pl.kernel,
    out_type=jax.ShapeDtypeStruct((NC, NPAD), jnp.float32),
    mesh=_mesh,
    scratch_types=[
        pltpu.VMEM((K, B), jnp.int32),
        pltpu.VMEM((B,), jnp.float32),
        pltpu.VMEM((RPT,), jnp.float32),
        pltpu.VMEM_SHARED((NPAD,), jnp.float32),
        pltpu.SemaphoreType.DMA,
    ],
    compiler_params=pltpu.CompilerParams(use_tc_tiling_on_sc=False),
)
def _deg_kernel(ei_hbm, out_hbm, dstv, onesv, zbuf, acc, sem):
    c = lax.axis_index("c")
    s = lax.axis_index("s")
    wid = s * NC + c
    one = jnp.ones((LANES,), jnp.float32)
    zero = jnp.zeros((LANES,), jnp.float32)
    for i in range(0, B, LANES):
        onesv[pl.ds(i, LANES)] = one
    for i in range(0, RPT, LANES):
        zbuf[pl.ds(i, LANES)] = zero
    pltpu.sync_copy(zbuf, acc.at[pl.ds(s * RPT, RPT)])
    pltpu.sync_copy(ei_hbm.at[1, wid], dstv)
    plsc.subcore_barrier()

    def fire(j, carry):
        pltpu.async_copy(onesv, acc.at[dstv.at[j]], sem, add=True)
        return carry

    lax.fori_loop(0, K, fire, 0)

    def drain(j, carry):
        pltpu.make_async_copy(onesv, acc.at[dstv.at[j]], sem).wait()
        return carry

    lax.fori_loop(0, K, drain, 0)
    plsc.subcore_barrier()
    pltpu.sync_copy(acc.at[pl.ds(s * RPT, RPT)],
                    out_hbm.at[c, pl.ds(s * RPT, RPT)])


NB = 8
GL = 6


def _edge_loop(hs_hbm, srcv, dstv, bufs, acc, gsem, ssem):
    for p in range(GL):
        pltpu.async_copy(hs_hbm.at[srcv.at[p]], bufs[p], gsem[p])

    def roundn(jo, carry):
        for r in range(NB):
            idx = NB * jo + r
            rn = (r + GL) % NB
            pltpu.make_async_copy(
                hs_hbm.at[srcv.at[idx]], bufs[r], gsem[r]).wait()
            pltpu.async_copy(
                bufs[r], acc.at[dstv.at[idx]], ssem[r], add=True)

            @pl.when(idx >= NB - GL)
            def _():
                pltpu.make_async_copy(
                    bufs[rn], acc.at[dstv.at[idx - (NB - GL)]],
                    ssem[rn]).wait()

            @pl.when(idx + GL < K)
            def _():
                pltpu.async_copy(
                    hs_hbm.at[srcv.at[idx + GL]], bufs[rn], gsem[rn])
        return carry

    lax.fori_loop(0, K // NB, roundn, 0)
    for t in range(K - (NB - GL), K):
        pltpu.make_async_copy(bufs[t % NB],
                              acc.at[dstv.at[t]], ssem[t % NB]).wait()


@functools.partial(
    pl.kernel,
    out_type=[jax.ShapeDtypeStruct((NC, NPAD, 32), jnp.float32),
              jax.ShapeDtypeStruct((NPAD, 32), jnp.float32),
              jax.ShapeDtypeStruct((NPAD, 32), jnp.float32),
              jax.ShapeDtypeStruct((NPAD // 2, LANES), jnp.float32)],
    mesh=_mesh,
    scratch_types=[
        pltpu.VMEM((K, B), jnp.int32),
        pltpu.VMEM((K, B), jnp.int32),
        [pltpu.VMEM((B, 32), jnp.float32)] * NB,
        pltpu.VMEM((RPT, 32), jnp.float32),
        pltpu.VMEM((RPT,), jnp.float32),
        pltpu.VMEM((RPT,), jnp.float32),
        pltpu.VMEM((RPT,), jnp.float32),
        pltpu.VMEM((RPT, 32), jnp.float32),
        pltpu.VMEM((RPT // 2, LANES), jnp.float32),
        pltpu.VMEM_SHARED((NPAD, 32), jnp.float32),
        [pltpu.SemaphoreType.DMA] * NB,
        [pltpu.SemaphoreType.DMA] * NB,
    ],
    compiler_params=pltpu.CompilerParams(use_tc_tiling_on_sc=False,
                                        needs_layout_passes=False),
)
def _layer1_kernel(h1_hbm, ei_hbm, degp_hbm,
                   out_hbm, hs1_hbm, dinv_hbm, dinv8_hbm,
                   srcv, dstv, bufs, hv, degv, degv2, dinvv, dinvb, dinvb8,
                   acc, gsem, ssem):
    c = lax.axis_index("c")
    s = lax.axis_index("s")
    wid = s * NC + c
    r0 = s * RPT
    zero = jnp.zeros((LANES,), jnp.float32)
    def zrow(r, carry):
        hv[r, pl.ds(0, LANES)] = zero
        hv[r, pl.ds(LANES, LANES)] = zero
        return carry

    lax.fori_loop(0, RPT, zrow, 0)
    pltpu.sync_copy(hv, acc.at[pl.ds(r0, RPT)])
    pltpu.sync_copy(ei_hbm.at[0, wid], srcv)
    pltpu.sync_copy(ei_hbm.at[1, wid], dstv)

    pltpu.sync_copy(degp_hbm.at[0, pl.ds(r0, RPT)], degv)
    pltpu.sync_copy(degp_hbm.at[1, pl.ds(r0, RPT)], degv2)
    magic = jnp.full((LANES,), 0x5F3759DF, jnp.int32)
    for i in range(0, RPT, LANES):
        d = degv[pl.ds(i, LANES)] + degv2[pl.ds(i, LANES)] + 1.0
        y = plsc.bitcast(
            magic - lax.shift_right_logical(plsc.bitcast(d, jnp.int32), 1),
            jnp.float32)
        for _ in range(3):
            y = y * (1.5 - 0.5 * d * y * y)
        dinvv[pl.ds(i, LANES)] = y

    pltpu.sync_copy(h1_hbm.at[pl.ds(r0, RPT)], hv)

    def scale_group(g, carry):
        dv = dinvv[pl.ds(g * LANES, LANES)]
        for j in range(LANES):
            r = g * LANES + j
            sc = dv[j]
            hv[r, pl.ds(0, LANES)] = hv[r, pl.ds(0, LANES)] * sc
            hv[r, pl.ds(LANES, LANES)] = hv[r, pl.ds(LANES, LANES)] * sc
            scb = jnp.full((LANES,), sc, jnp.float32)
            dinvb[r, pl.ds(0, LANES)] = scb
            dinvb[r, pl.ds(LANES, LANES)] = scb
        return carry

    lax.fori_loop(0, RPT // LANES, scale_group, 0)

    ge8 = (lax.iota(jnp.int32, LANES) >= 8).astype(jnp.int32)

    def pair_group(g, carry):
        idx = 2 * g + ge8
        dinvb8[g, pl.ds(0, LANES)] = plsc.load_gather(dinvv, [idx])
        return carry

    lax.fori_loop(0, RPT // 2, pair_group, 0)
    pltpu.sync_copy(dinvb, dinv_hbm.at[pl.ds(r0, RPT)])
    pltpu.sync_copy(dinvb8, dinv8_hbm.at[pl.ds(s * (RPT // 2), RPT // 2)])
    pltpu.sync_copy(hv, hs1_hbm.at[pl.ds(r0, RPT)])
    plsc.subcore_barrier()

    _edge_loop(hs1_hbm, srcv, dstv, bufs, acc, gsem, ssem)
    plsc.subcore_barrier()
    pltpu.sync_copy(acc.at[pl.ds(r0, RPT)], out_hbm.at[c, pl.ds(r0, RPT)])


@functools.partial(
    pl.kernel,
    out_type=jax.ShapeDtypeStruct((NC, NPAD, 8), jnp.float32),
    mesh=_mesh,
    scratch_types=[
        pltpu.VMEM((K, B), jnp.int32),
        pltpu.VMEM((K, B), jnp.int32),
        [pltpu.VMEM((B, 8), jnp.float32)] * NB,
        pltpu.VMEM_SHARED((NPAD, 8), jnp.float32),
        [pltpu.SemaphoreType.DMA] * NB,
        [pltpu.SemaphoreType.DMA] * NB,
    ],
    compiler_params=pltpu.CompilerParams(use_tc_tiling_on_sc=False),
)
def _agg8(hs_hbm, ei_hbm, zer_hbm, out_hbm,
          srcv, dstv, bufs, acc, gsem, ssem):
    c = lax.axis_index("c")
    s = lax.axis_index("s")
    wid = s * NC + c
    r0 = s * RPT
    pltpu.sync_copy(zer_hbm.at[pl.ds(r0, RPT)], acc.at[pl.ds(r0, RPT)])
    pltpu.sync_copy(ei_hbm.at[0, wid], srcv)
    pltpu.sync_copy(ei_hbm.at[1, wid], dstv)
    plsc.subcore_barrier()
    _edge_loop(hs_hbm, srcv, dstv, bufs, acc, gsem, ssem)
    plsc.subcore_barrier()
    pltpu.sync_copy(acc.at[pl.ds(r0, RPT)], out_hbm.at[c, pl.ds(r0, RPT)])


_RB = 1024
_GRID = NPAD // _RB


def _k2a_body(x_ref, w1_ref, h1_ref):
    h1_ref[...] = jnp.dot(x_ref[...], w1_ref[...],
                          preferred_element_type=jnp.float32)


def _k4_body(p_ref, hs1_ref, dinv_ref, b1_ref, w2d_ref, hs2_ref):
    dinv = dinv_ref[...]
    out1 = (p_ref[0] + p_ref[1] + hs1_ref[...]) * dinv + b1_ref[...]
    h = jnp.maximum(out1, 0.0) * dinv
    hs2_ref[...] = jnp.dot(h, w2d_ref[...],
                           preferred_element_type=jnp.float32)


def _k6_body(p_ref, hs2_ref, dinv_ref, b2_ref, out_ref):
    logits = ((p_ref[0] + p_ref[1] + hs2_ref[...]) * dinv_ref[...]
              + b2_ref[...])
    for g in range(16):
        sl = logits[:, 8 * g:8 * g + 8]
        m = jnp.max(sl, axis=1, keepdims=True)
        e = jnp.exp(sl - m)
        out_ref[:, 8 * g:8 * g + 8] = e / jnp.sum(e, axis=1, keepdims=True)


@jax.jit
def kernel(x, edge_index, W1, b1, W2, b2):
    f32 = jnp.float32
    ei = edge_index.astype(jnp.int32)
    npad_e = EPAD - E
    pad_i = jnp.arange(npad_e, dtype=jnp.int32)
    pads = jnp.stack([pad_i % N, N + pad_i % (NPAD - N)])
    eiT = jnp.concatenate([ei, pads], axis=1).reshape(2, NT, K, B)

    xp = jnp.concatenate([x, jnp.zeros((NPAD - N, x.shape[1]), f32)])
    w2p = jnp.concatenate([W2, jnp.zeros((W2.shape[0], 1), f32)], axis=1)
    b1r = b1.reshape(1, -1)
    b2r = jnp.concatenate([b2, jnp.full((1,), -1e30, f32)]).reshape(1, 8)
    zer8 = jnp.zeros((NPAD, 8), f32)

    degp = _deg_kernel(eiT)
    h1 = pl.pallas_call(
        _k2a_body,
        grid=(_GRID,),
        in_specs=[pl.BlockSpec((_RB, 128), lambda i: (i, 0)),
                  pl.BlockSpec((128, 32), lambda i: (0, 0))],
        out_specs=pl.BlockSpec((_RB, 32), lambda i: (i, 0)),
        out_shape=jax.ShapeDtypeStruct((NPAD, 32), f32),
    )(xp, W1)
    p1, hs1, dinv32, dinv8 = _layer1_kernel(h1, eiT, degp)

    n4 = NPAD // 4
    rb4 = _RB // 4
    hs2p = pl.pallas_call(
        _k4_body,
        grid=(_GRID,),
        in_specs=[pl.BlockSpec((NC, rb4, 128), lambda i: (0, i, 0)),
                  pl.BlockSpec((rb4, 128), lambda i: (i, 0)),
                  pl.BlockSpec((rb4, 128), lambda i: (i, 0)),
                  pl.BlockSpec((1, 128), lambda i: (0, 0)),
                  pl.BlockSpec((128, 32), lambda i: (0, 0))],
        out_specs=pl.BlockSpec((rb4, 32), lambda i: (i, 0)),
        out_shape=jax.ShapeDtypeStruct((n4, 32), f32),
    )(p1.reshape(NC, n4, 128), hs1.reshape(n4, 128),
      dinv32.reshape(n4, 128), jnp.tile(b1r, (1, 4)),
      jnp.kron(jnp.eye(4, dtype=f32), w2p))
    hs2 = hs2p.reshape(NPAD, 8)

    p2 = _agg8(hs2, eiT, zer8)
    n16 = NPAD // 16
    rb16 = _RB // 16
    outp = pl.pallas_call(
        _k6_body,
        grid=(_GRID,),
        in_specs=[pl.BlockSpec((NC, rb16, 128), lambda i: (0, i, 0)),
                  pl.BlockSpec((rb16, 128), lambda i: (i, 0)),
                  pl.BlockSpec((rb16, 128), lambda i: (i, 0)),
                  pl.BlockSpec((1, 128), lambda i: (0, 0))],
        out_specs=pl.BlockSpec((rb16, 128), lambda i: (i, 0)),
        out_shape=jax.ShapeDtypeStruct((n16, 128), f32),
    )(p2.reshape(NC, n16, 128), hs2.reshape(n16, 128),
      dinv8.reshape(n16, 128), jnp.tile(b2r, (1, 16)))
    return outp.reshape(NPAD, 8)[:N, :7]
